# baseline (speedup 1.0000x reference)
import functools
import math

import jax
import jax.numpy as jnp
from jax import lax
from jax.experimental import pallas as pl
from jax.experimental.pallas import tpu as pltpu
from jax.experimental.pallas import tpu_sc as plsc

N = 10000
E = 320000
D_ATTR = 16
D_FEAT = 128
D_EFEAT = 8

NC = 2
NS = 16
NW = NC * NS

C = 128
NCHUNK = E // C
MAIN = NCHUNK // NW
EXTRA = NCHUNK - MAIN * NW
TROW = E // 16
N_PAD = 10240
ROWS_PER_TILE = N_PAD // NS

_INV_SQRT_F = float(1.0 / math.sqrt(D_FEAT))
_INV_SQRT_E = float(1.0 / math.sqrt(D_EFEAT))
_INV_SQRT_SKIP = float(1.0 / math.sqrt(D_FEAT * D_ATTR))
_LOG2 = float(math.log(2.0))


def _h_body(nf_ref, w1_ref, out_ref):
    out_ref[...] = jnp.dot(nf_ref[...], w1_ref[...],
                           preferred_element_type=jnp.float32) * _INV_SQRT_F


def _compute_h(node_feats, W1):
    return pl.pallas_call(
        _h_body,
        out_shape=jax.ShapeDtypeStruct((N, D_FEAT), jnp.float32),
    )(node_feats, W1)


_EBLK = 16000


def _t2_body(efT_ref, eaT_ref, wm1T_ref, out_ref):
    pre = jnp.dot(wm1T_ref[...], efT_ref[...],
                  preferred_element_type=jnp.float32)
    t = jax.nn.softplus(pre * _INV_SQRT_E) - _LOG2
    out_ref[...] = t * eaT_ref[...] * _INV_SQRT_E


def _compute_t2w(efT, eaT, Wm1T):
    grid = (E // _EBLK,)
    return pl.pallas_call(
        _t2_body,
        grid=grid,
        in_specs=[
            pl.BlockSpec((D_EFEAT, _EBLK), lambda i: (0, i)),
            pl.BlockSpec((1, _EBLK), lambda i: (0, i)),
            pl.BlockSpec((D_EFEAT, D_EFEAT), lambda i: (0, 0)),
        ],
        out_specs=pl.BlockSpec((D_EFEAT, _EBLK), lambda i: (0, i)),
        out_shape=jax.ShapeDtypeStruct((D_EFEAT, E), jnp.float32),
    )(efT, eaT, Wm1T)


def _sc_body(h_hbm, t2w_hbm, eidx_hbm, zeros_hbm, wm2_hbm, out_hbm,
             sidx0, sidx1, ridx0, ridx1, srdx0, srdx1, rows0, rows1,
             t20, t21, wm2_v, m_shared,
             semA, semB, semI0, semI1, semS0, semS1):
    cid = lax.axis_index("c")
    sid = lax.axis_index("s")
    wid = sid * NC + cid
    qbase = wid * MAIN

    pltpu.sync_copy(wm2_hbm, wm2_v)
    pltpu.sync_copy(zeros_hbm, m_shared.at[pl.ds(sid * ROWS_PER_TILE,
                                                 ROWS_PER_TILE)])
    plsc.subcore_barrier()

    sidx_bufs = (sidx0, sidx1)
    ridx_bufs = (ridx0, ridx1)
    srdx_bufs = (srdx0, srdx1)
    rows_bufs = (rows0, rows1)
    t2_bufs = (t20, t21)
    sems = (semA, semB)
    semsI = (semI0, semI1)
    semsS = (semS0, semS1)

    def start_idx(q, par):
        pltpu.async_copy(eidx_hbm.at[0, pl.ds(q * C, C)], sidx_bufs[par],
                         semsI[par])
        pltpu.async_copy(eidx_hbm.at[1, pl.ds(q * C, C)], ridx_bufs[par],
                         semsI[par])
        pltpu.async_copy(t2w_hbm.at[:, pl.ds(q * C, C)], t2_bufs[par],
                         semsI[par])

    def wait_idx(par):
        pltpu.make_async_copy(eidx_hbm.at[0, pl.ds(0, C)], sidx_bufs[par],
                              semsI[par]).wait()
        pltpu.make_async_copy(eidx_hbm.at[1, pl.ds(0, C)], ridx_bufs[par],
                              semsI[par]).wait()
        pltpu.make_async_copy(t2w_hbm.at[:, pl.ds(0, C)], t2_bufs[par],
                              semsI[par]).wait()

    def start_gather(par):
        pltpu.async_copy(h_hbm.at[sidx_bufs[par]], rows_bufs[par], sems[par])

    def wait_gather(par):
        pltpu.make_async_copy(h_hbm.at[pl.ds(0, C)], rows_bufs[par],
                              sems[par]).wait()

    def start_scatter(par):
        for g in range(8):
            sl = pl.ds(g * 16, 16)
            srdx_bufs[par][sl] = ridx_bufs[par][sl]
        pltpu.async_copy(rows_bufs[par], m_shared.at[srdx_bufs[par]],
                         semsS[par], add=True)

    def wait_scatter(par):
        pltpu.make_async_copy(h_hbm.at[pl.ds(0, C)], rows_bufs[par],
                              semsS[par]).wait()

    def process(par):
        rows_v = rows_bufs[par]
        t2_v = t2_bufs[par]

        def grp16(r, c):
            tvs = [t2_v[k, pl.ds(r * 16, 16)] for k in range(D_EFEAT)]
            tse = [[tvs[k][eo] for k in range(D_EFEAT)] for eo in range(16)]
            for cg in range(8):
                wv = [wm2_v[k, pl.ds(cg * 16, 16)] for k in range(D_EFEAT)]
                sl = pl.ds(cg * 16, 16)
                for eo in range(16):
                    i = r * 16 + eo
                    ts = tse[eo]
                    acc = wv[0] * ts[0]
                    for k in range(1, D_EFEAT):
                        acc = acc + wv[k] * ts[k]
                    rows_v[i, sl] = rows_v[i, sl] * acc
            return c

        lax.fori_loop(0, C // 16, grp16, 0, unroll=False)

    start_idx(qbase, 0)
    wait_idx(0)
    start_gather(0)
    start_idx(qbase + 1, 1)

    def loop(u, carry):
        for b in range(2):
            j = 2 * u + b
            nxt = 1 - b

            @pl.when(j + 1 < MAIN)
            def _():
                @pl.when(j >= 1)
                def _():
                    wait_scatter(nxt)
                wait_idx(nxt)
                start_gather(nxt)

            wait_gather(b)
            process(b)
            start_scatter(b)

            @pl.when(j + 2 < MAIN)
            def _():
                start_idx(qbase + j + 2, b)
        return carry

    lax.fori_loop(0, MAIN // 2, loop, 0, unroll=False)
    wait_scatter(0)
    wait_scatter(1)

    @pl.when(wid < EXTRA)
    def _():
        q = NW * MAIN + wid
        start_idx(q, 0)
        wait_idx(0)
        start_gather(0)
        wait_gather(0)
        process(0)
        pltpu.sync_copy(rows0, m_shared.at[ridx0], add=True)

    plsc.subcore_barrier()

    pltpu.sync_copy(m_shared.at[pl.ds(sid * ROWS_PER_TILE, ROWS_PER_TILE)],
                    out_hbm.at[cid, pl.ds(sid * ROWS_PER_TILE, ROWS_PER_TILE)])


def _sc_scatter(h, t2w, edge_index, zeros_slab, Wm2):
    mesh = plsc.VectorSubcoreMesh(core_axis_name="c", subcore_axis_name="s")
    fn = functools.partial(
        pl.kernel,
        out_type=jax.ShapeDtypeStruct((NC, N_PAD, D_FEAT), jnp.float32),
        mesh=mesh,
        scratch_types=[
            pltpu.VMEM((C,), jnp.int32),
            pltpu.VMEM((C,), jnp.int32),
            pltpu.VMEM((C,), jnp.int32),
            pltpu.VMEM((C,), jnp.int32),
            pltpu.VMEM((C,), jnp.int32),
            pltpu.VMEM((C,), jnp.int32),
            pltpu.VMEM((C, D_FEAT), jnp.float32),
            pltpu.VMEM((C, D_FEAT), jnp.float32),
            pltpu.VMEM((8, 128), jnp.float32),
            pltpu.VMEM((8, 128), jnp.float32),
            pltpu.VMEM((D_EFEAT, D_FEAT), jnp.float32),
            pltpu.VMEM_SHARED((N_PAD, D_FEAT), jnp.float32),
            pltpu.SemaphoreType.DMA,
            pltpu.SemaphoreType.DMA,
            pltpu.SemaphoreType.DMA,
            pltpu.SemaphoreType.DMA,
            pltpu.SemaphoreType.DMA,
            pltpu.SemaphoreType.DMA,
        ],
    )(_sc_body)
    return fn(h, t2w, edge_index, zeros_slab, Wm2)


_NBLK = 1000


def _final_body(mp_ref, attrs_ref, w2_ref, wskipT_ref, out_ref):
    m = (mp_ref[0] + mp_ref[1]) @ w2_ref[...] * _INV_SQRT_F
    acc = m
    a = attrs_ref[...]
    for v in range(D_ATTR):
        acc = acc + jnp.dot(m * a[:, v:v + 1], wskipT_ref[v],
                            preferred_element_type=jnp.float32) * _INV_SQRT_SKIP
    out_ref[...] = acc


def _final(mpart, node_attrs, W2, WskipT):
    grid = (N // _NBLK,)
    return pl.pallas_call(
        _final_body,
        grid=grid,
        in_specs=[
            pl.BlockSpec((NC, _NBLK, D_FEAT), lambda i: (0, i, 0)),
            pl.BlockSpec((_NBLK, D_ATTR), lambda i: (i, 0)),
            pl.BlockSpec((D_FEAT, D_FEAT), lambda i: (0, 0)),
            pl.BlockSpec((D_ATTR, D_FEAT, D_FEAT), lambda i: (0, 0, 0)),
        ],
        out_specs=pl.BlockSpec((_NBLK, D_FEAT), lambda i: (i, 0)),
        out_shape=jax.ShapeDtypeStruct((N, D_FEAT), jnp.float32),
    )(mpart, node_attrs, W2, WskipT)


def kernel(node_attrs, node_feats, edge_attrs, edge_feats, edge_index,
           W1, Wm1, Wm2, W2, Wskip):
    zeros_slab = jnp.zeros((ROWS_PER_TILE, D_FEAT), jnp.float32)
    WskipT = jnp.transpose(Wskip, (1, 0, 2))

    h = _compute_h(node_feats, W1)
    t2w = _compute_t2w(edge_feats.T, edge_attrs.T, Wm1.T)
    mpart = _sc_scatter(h, t2w, edge_index, zeros_slab, Wm2)
    return _final(mpart, node_attrs, W2, WskipT)


# speedup vs baseline: 1.1580x; 1.1580x over previous
import functools
import math

import jax
import jax.numpy as jnp
from jax import lax
from jax.experimental import pallas as pl
from jax.experimental.pallas import tpu as pltpu
from jax.experimental.pallas import tpu_sc as plsc

N = 10000
E = 320000
D_ATTR = 16
D_FEAT = 128
D_EFEAT = 8

NC = 2
NS = 16
NW = NC * NS

C = 128
NCHUNK = E // C
MAIN = NCHUNK // NW
EXTRA = NCHUNK - MAIN * NW
TROW = E // 16
N_PAD = 10240
ROWS_PER_TILE = N_PAD // NS

_INV_SQRT_F = float(1.0 / math.sqrt(D_FEAT))
_INV_SQRT_E = float(1.0 / math.sqrt(D_EFEAT))
_INV_SQRT_SKIP = float(1.0 / math.sqrt(D_FEAT * D_ATTR))
_LOG2 = float(math.log(2.0))


def _h_body(nf_ref, w1_ref, out_ref):
    out_ref[...] = jnp.dot(nf_ref[...], w1_ref[...],
                           preferred_element_type=jnp.float32) * _INV_SQRT_F


def _compute_h(node_feats, W1):
    return pl.pallas_call(
        _h_body,
        out_shape=jax.ShapeDtypeStruct((N, D_FEAT), jnp.float32),
    )(node_feats, W1)


_EBLK = 16000


def _t2_body(efT_ref, eaT_ref, wm1T_ref, out_ref):
    pre = jnp.dot(wm1T_ref[...], efT_ref[...],
                  preferred_element_type=jnp.float32)
    t = jax.nn.softplus(pre * _INV_SQRT_E) - _LOG2
    out_ref[...] = t * eaT_ref[...] * _INV_SQRT_E


def _compute_t2w(efT, eaT, Wm1T):
    grid = (E // _EBLK,)
    return pl.pallas_call(
        _t2_body,
        grid=grid,
        in_specs=[
            pl.BlockSpec((D_EFEAT, _EBLK), lambda i: (0, i)),
            pl.BlockSpec((1, _EBLK), lambda i: (0, i)),
            pl.BlockSpec((D_EFEAT, D_EFEAT), lambda i: (0, 0)),
        ],
        out_specs=pl.BlockSpec((D_EFEAT, _EBLK), lambda i: (0, i)),
        out_shape=jax.ShapeDtypeStruct((D_EFEAT, E), jnp.float32),
    )(efT, eaT, Wm1T)


def _sc_body(h_hbm, t2w_hbm, eidx_hbm, zeros_hbm, wm2_hbm, out_hbm,
             sidx0, sidx1, ridx0, ridx1, srdx0, srdx1, rows0, rows1,
             t20, t21, wm2_v, m_shared,
             semA, semB, semI0, semI1, semS0, semS1):
    cid = lax.axis_index("c")
    sid = lax.axis_index("s")
    wid = sid * NC + cid
    qbase = wid * MAIN

    pltpu.sync_copy(wm2_hbm, wm2_v)
    pltpu.sync_copy(zeros_hbm, m_shared.at[pl.ds(sid * ROWS_PER_TILE,
                                                 ROWS_PER_TILE)])
    plsc.subcore_barrier()

    sidx_bufs = (sidx0, sidx1)
    ridx_bufs = (ridx0, ridx1)
    srdx_bufs = (srdx0, srdx1)
    rows_bufs = (rows0, rows1)
    t2_bufs = (t20, t21)
    sems = (semA, semB)
    semsI = (semI0, semI1)
    semsS = (semS0, semS1)

    def start_idx(q, par):
        pltpu.async_copy(eidx_hbm.at[0, pl.ds(q * C, C)], sidx_bufs[par],
                         semsI[par])
        pltpu.async_copy(eidx_hbm.at[1, pl.ds(q * C, C)], ridx_bufs[par],
                         semsI[par])
        pltpu.async_copy(t2w_hbm.at[:, pl.ds(q * C, C)], t2_bufs[par],
                         semsI[par])

    def wait_idx(par):
        pltpu.make_async_copy(eidx_hbm.at[0, pl.ds(0, C)], sidx_bufs[par],
                              semsI[par]).wait()
        pltpu.make_async_copy(eidx_hbm.at[1, pl.ds(0, C)], ridx_bufs[par],
                              semsI[par]).wait()
        pltpu.make_async_copy(t2w_hbm.at[:, pl.ds(0, C)], t2_bufs[par],
                              semsI[par]).wait()

    def start_gather(par):
        pltpu.async_copy(h_hbm.at[sidx_bufs[par]], rows_bufs[par], sems[par])

    def wait_gather(par):
        pltpu.make_async_copy(h_hbm.at[pl.ds(0, C)], rows_bufs[par],
                              sems[par]).wait()

    def start_scatter(par):
        for g in range(8):
            sl = pl.ds(g * 16, 16)
            srdx_bufs[par][sl] = ridx_bufs[par][sl]
        pltpu.async_copy(rows_bufs[par], m_shared.at[srdx_bufs[par]],
                         semsS[par], add=True)

    def wait_scatter(par):
        pltpu.make_async_copy(h_hbm.at[pl.ds(0, C)], rows_bufs[par],
                              semsS[par]).wait()

    def process(par):
        rows_v = rows_bufs[par]
        t2_v = t2_bufs[par]

        for half in range(2):
            wv = [[wm2_v[k, pl.ds(half * 64 + cg * 16, 16)]
                   for k in range(D_EFEAT)] for cg in range(4)]

            def grp16(r, c):
                tvs = [t2_v[k, pl.ds(r * 16, 16)] for k in range(D_EFEAT)]
                for eo in range(16):
                    i = r * 16 + eo
                    ts = [tvs[k][eo] for k in range(D_EFEAT)]
                    for cg in range(4):
                        acc = wv[cg][0] * ts[0]
                        for k in range(1, D_EFEAT):
                            acc = acc + wv[cg][k] * ts[k]
                        sl = pl.ds(half * 64 + cg * 16, 16)
                        rows_v[i, sl] = rows_v[i, sl] * acc
                return c

            lax.fori_loop(0, C // 16, grp16, 0, unroll=False)

    start_idx(qbase, 0)
    wait_idx(0)
    start_gather(0)
    start_idx(qbase + 1, 1)

    def loop(u, carry):
        for b in range(2):
            j = 2 * u + b
            nxt = 1 - b

            @pl.when(j + 1 < MAIN)
            def _():
                @pl.when(j >= 1)
                def _():
                    wait_scatter(nxt)
                wait_idx(nxt)
                start_gather(nxt)

            wait_gather(b)
            process(b)
            start_scatter(b)

            @pl.when(j + 2 < MAIN)
            def _():
                start_idx(qbase + j + 2, b)
        return carry

    lax.fori_loop(0, MAIN // 2, loop, 0, unroll=False)
    wait_scatter(0)
    wait_scatter(1)

    @pl.when(wid < EXTRA)
    def _():
        q = NW * MAIN + wid
        start_idx(q, 0)
        wait_idx(0)
        start_gather(0)
        wait_gather(0)
        process(0)
        pltpu.sync_copy(rows0, m_shared.at[ridx0], add=True)

    plsc.subcore_barrier()

    pltpu.sync_copy(m_shared.at[pl.ds(sid * ROWS_PER_TILE, ROWS_PER_TILE)],
                    out_hbm.at[cid, pl.ds(sid * ROWS_PER_TILE, ROWS_PER_TILE)])


def _sc_scatter(h, t2w, edge_index, zeros_slab, Wm2):
    mesh = plsc.VectorSubcoreMesh(core_axis_name="c", subcore_axis_name="s")
    fn = functools.partial(
        pl.kernel,
        out_type=jax.ShapeDtypeStruct((NC, N_PAD, D_FEAT), jnp.float32),
        mesh=mesh,
        scratch_types=[
            pltpu.VMEM((C,), jnp.int32),
            pltpu.VMEM((C,), jnp.int32),
            pltpu.VMEM((C,), jnp.int32),
            pltpu.VMEM((C,), jnp.int32),
            pltpu.VMEM((C,), jnp.int32),
            pltpu.VMEM((C,), jnp.int32),
            pltpu.VMEM((C, D_FEAT), jnp.float32),
            pltpu.VMEM((C, D_FEAT), jnp.float32),
            pltpu.VMEM((8, 128), jnp.float32),
            pltpu.VMEM((8, 128), jnp.float32),
            pltpu.VMEM((D_EFEAT, D_FEAT), jnp.float32),
            pltpu.VMEM_SHARED((N_PAD, D_FEAT), jnp.float32),
            pltpu.SemaphoreType.DMA,
            pltpu.SemaphoreType.DMA,
            pltpu.SemaphoreType.DMA,
            pltpu.SemaphoreType.DMA,
            pltpu.SemaphoreType.DMA,
            pltpu.SemaphoreType.DMA,
        ],
    )(_sc_body)
    return fn(h, t2w, edge_index, zeros_slab, Wm2)


_NBLK = 1000


def _final_body(mp_ref, attrs_ref, w2_ref, wskipT_ref, out_ref):
    m = (mp_ref[0] + mp_ref[1]) @ w2_ref[...] * _INV_SQRT_F
    acc = m
    a = attrs_ref[...]
    for v in range(D_ATTR):
        acc = acc + jnp.dot(m * a[:, v:v + 1], wskipT_ref[v],
                            preferred_element_type=jnp.float32) * _INV_SQRT_SKIP
    out_ref[...] = acc


def _final(mpart, node_attrs, W2, WskipT):
    grid = (N // _NBLK,)
    return pl.pallas_call(
        _final_body,
        grid=grid,
        in_specs=[
            pl.BlockSpec((NC, _NBLK, D_FEAT), lambda i: (0, i, 0)),
            pl.BlockSpec((_NBLK, D_ATTR), lambda i: (i, 0)),
            pl.BlockSpec((D_FEAT, D_FEAT), lambda i: (0, 0)),
            pl.BlockSpec((D_ATTR, D_FEAT, D_FEAT), lambda i: (0, 0, 0)),
        ],
        out_specs=pl.BlockSpec((_NBLK, D_FEAT), lambda i: (i, 0)),
        out_shape=jax.ShapeDtypeStruct((N, D_FEAT), jnp.float32),
    )(mpart, node_attrs, W2, WskipT)


def kernel(node_attrs, node_feats, edge_attrs, edge_feats, edge_index,
           W1, Wm1, Wm2, W2, Wskip):
    zeros_slab = jnp.zeros((ROWS_PER_TILE, D_FEAT), jnp.float32)
    WskipT = jnp.transpose(Wskip, (1, 0, 2))

    h = _compute_h(node_feats, W1)
    t2w = _compute_t2w(edge_feats.T, edge_attrs.T, Wm1.T)
    mpart = _sc_scatter(h, t2w, edge_index, zeros_slab, Wm2)
    return _final(mpart, node_attrs, W2, WskipT)
